# baseline (device time: 14501 ns/iter reference)
import jax
import jax.numpy as jnp
from jax import lax
from jax.experimental import pallas as pl
from jax.experimental.pallas import tpu as pltpu

N_DEV = 4
EPS = 1e-5


def kernel(x, gamma, beta):
    m, n_local = x.shape
    n_global = N_DEV * n_local
    gamma2 = gamma.reshape(1, n_local)
    beta2 = beta.reshape(1, n_local)

    def body(x_ref, g_ref, b_ref, o_ref, acc_ref, comm_ref, send_sems, recv_sems):
        my = lax.axis_index("i")
        p0 = my ^ 1
        p1 = my ^ 2

        barrier_sem = pltpu.get_barrier_semaphore()
        for p in (p0, p1):
            pl.semaphore_signal(
                barrier_sem, inc=1,
                device_id=(p,), device_id_type=pl.DeviceIdType.MESH,
            )
        pl.semaphore_wait(barrier_sem, 2)

        xv = x_ref[:, :]
        s = jnp.sum(xv, axis=1, keepdims=True)
        sq = jnp.sum(xv * xv, axis=1, keepdims=True)
        acc_ref[0] = jnp.concatenate([s, sq], axis=1)

        for stage, partner in ((0, p0), (1, p1)):
            rdma = pltpu.make_async_remote_copy(
                src_ref=acc_ref.at[stage],
                dst_ref=comm_ref.at[stage],
                send_sem=send_sems.at[stage],
                recv_sem=recv_sems.at[stage],
                device_id=(partner,),
                device_id_type=pl.DeviceIdType.MESH,
            )
            rdma.start()
            rdma.wait()
            acc_ref[stage + 1] = acc_ref[stage] + comm_ref[stage]

        total = acc_ref[2]
        mean = total[:, 0:1] * (1.0 / n_global)
        var = total[:, 1:2] * (1.0 / n_global) - mean * mean
        inv = lax.rsqrt(var + EPS)
        o_ref[:, :] = g_ref[:, :] * ((xv - mean) * inv) + b_ref[:, :]

    return pl.pallas_call(
        body,
        out_shape=jax.ShapeDtypeStruct((m, n_local), x.dtype),
        in_specs=[
            pl.BlockSpec(memory_space=pltpu.VMEM),
            pl.BlockSpec(memory_space=pltpu.VMEM),
            pl.BlockSpec(memory_space=pltpu.VMEM),
        ],
        out_specs=pl.BlockSpec(memory_space=pltpu.VMEM),
        scratch_shapes=[
            pltpu.VMEM((3, m, 2), jnp.float32),
            pltpu.VMEM((2, m, 2), jnp.float32),
            pltpu.SemaphoreType.DMA((2,)),
            pltpu.SemaphoreType.DMA((2,)),
        ],
        compiler_params=pltpu.CompilerParams(collective_id=0),
    )(x, gamma2, beta2)


# device time: 12560 ns/iter; 1.1545x vs baseline; 1.1545x over previous
import jax
import jax.numpy as jnp
from jax import lax
from jax.experimental import pallas as pl
from jax.experimental.pallas import tpu as pltpu

N_DEV = 4
EPS = 1e-5


def kernel(x, gamma, beta):
    m, n_local = x.shape
    n_global = N_DEV * n_local
    gamma2 = gamma.reshape(1, n_local)
    beta2 = beta.reshape(1, n_local)

    def body(x_ref, g_ref, b_ref, o_ref, acc_ref, comm_ref, send_sems, recv_sems):
        my = lax.axis_index("i")

        xv = x_ref[:, :]
        s = jnp.sum(xv, axis=1, keepdims=True)
        sq = jnp.sum(xv * xv, axis=1, keepdims=True)
        acc_ref[:, :] = jnp.concatenate([s, sq], axis=1)

        barrier_sem = pltpu.get_barrier_semaphore()
        for k in (1, 2, 3):
            pl.semaphore_signal(
                barrier_sem, inc=1,
                device_id=(my ^ k,), device_id_type=pl.DeviceIdType.MESH,
            )
        pl.semaphore_wait(barrier_sem, 3)

        rdmas = []
        for k in (1, 2, 3):
            rdma = pltpu.make_async_remote_copy(
                src_ref=acc_ref,
                dst_ref=comm_ref.at[k - 1],
                send_sem=send_sems.at[k - 1],
                recv_sem=recv_sems.at[k - 1],
                device_id=(my ^ k,),
                device_id_type=pl.DeviceIdType.MESH,
            )
            rdma.start()
            rdmas.append(rdma)

        xg = xv * g_ref[:, :]

        for rdma in rdmas:
            rdma.wait_recv()
        total = acc_ref[:, :] + comm_ref[0] + comm_ref[1] + comm_ref[2]

        mean = total[:, 0:1] * (1.0 / n_global)
        var = total[:, 1:2] * (1.0 / n_global) - mean * mean
        inv = lax.rsqrt(var + EPS)
        o_ref[:, :] = xg * inv - g_ref[:, :] * (mean * inv) + b_ref[:, :]

        for rdma in rdmas:
            rdma.wait_send()

    return pl.pallas_call(
        body,
        out_shape=jax.ShapeDtypeStruct((m, n_local), x.dtype),
        in_specs=[
            pl.BlockSpec(memory_space=pltpu.VMEM),
            pl.BlockSpec(memory_space=pltpu.VMEM),
            pl.BlockSpec(memory_space=pltpu.VMEM),
        ],
        out_specs=pl.BlockSpec(memory_space=pltpu.VMEM),
        scratch_shapes=[
            pltpu.VMEM((m, 2), jnp.float32),
            pltpu.VMEM((3, m, 2), jnp.float32),
            pltpu.SemaphoreType.DMA((3,)),
            pltpu.SemaphoreType.DMA((3,)),
        ],
        compiler_params=pltpu.CompilerParams(collective_id=0),
    )(x, gamma2, beta2)


# device time: 7806 ns/iter; 1.8577x vs baseline; 1.6090x over previous
import jax
import jax.numpy as jnp
from jax import lax
from jax.experimental import pallas as pl
from jax.experimental.pallas import tpu as pltpu

N_DEV = 4
EPS = 1e-5


def kernel(x, gamma, beta):
    m, n_local = x.shape
    n_global = N_DEV * n_local
    gamma2 = gamma.reshape(1, n_local)
    beta2 = beta.reshape(1, n_local)

    def body(x_ref, g_ref, b_ref, o_ref, acc_ref, comm_ref, send_sems, recv_sems):
        my = lax.axis_index("i")

        barrier_sem = pltpu.get_barrier_semaphore()
        for k in (1, 2, 3):
            pl.semaphore_signal(
                barrier_sem, inc=1,
                device_id=(my ^ k,), device_id_type=pl.DeviceIdType.MESH,
            )

        xv = x_ref[:, :]
        s = jnp.sum(xv, axis=1)
        sq = jnp.sum(xv * xv, axis=1)
        acc_ref[:, :] = jnp.stack([s, sq])

        pl.semaphore_wait(barrier_sem, 3)

        rdmas = []
        for k in (1, 2, 3):
            rdma = pltpu.make_async_remote_copy(
                src_ref=acc_ref,
                dst_ref=comm_ref.at[k - 1],
                send_sem=send_sems.at[k - 1],
                recv_sem=recv_sems.at[k - 1],
                device_id=(my ^ k,),
                device_id_type=pl.DeviceIdType.MESH,
            )
            rdma.start()
            rdmas.append(rdma)

        xg = xv * g_ref[:, :]

        for rdma in rdmas:
            rdma.wait_recv()
        total = acc_ref[:, :] + comm_ref[0] + comm_ref[1] + comm_ref[2]

        mean = total[0, :].reshape(m, 1) * (1.0 / n_global)
        var = total[1, :].reshape(m, 1) * (1.0 / n_global) - mean * mean
        inv = lax.rsqrt(var + EPS)
        o_ref[:, :] = xg * inv - g_ref[:, :] * (mean * inv) + b_ref[:, :]

        for rdma in rdmas:
            rdma.wait_send()

    return pl.pallas_call(
        body,
        out_shape=jax.ShapeDtypeStruct((m, n_local), x.dtype),
        in_specs=[
            pl.BlockSpec(memory_space=pltpu.VMEM),
            pl.BlockSpec(memory_space=pltpu.VMEM),
            pl.BlockSpec(memory_space=pltpu.VMEM),
        ],
        out_specs=pl.BlockSpec(memory_space=pltpu.VMEM),
        scratch_shapes=[
            pltpu.VMEM((2, m), jnp.float32),
            pltpu.VMEM((3, 2, m), jnp.float32),
            pltpu.SemaphoreType.DMA((3,)),
            pltpu.SemaphoreType.DMA((3,)),
        ],
        compiler_params=pltpu.CompilerParams(collective_id=0),
    )(x, gamma2, beta2)
